# baseline (speedup 1.0000x reference)
import functools

import jax
import jax.numpy as jnp
from jax.experimental import pallas as pl
from jax.experimental.pallas import tpu as pltpu

EMBD = 128
B1 = 32


@functools.cache
def _make_tc(n_total: int, vocab: int):
    n_rows = n_total // 128
    grid = n_rows // B1
    assert grid * B1 * 128 == n_total

    def body(idx_ref, table_ref, out_ref):
        idx = idx_ref[...]
        iota = jax.lax.broadcasted_iota(jnp.int32, (B1, 128, vocab), 2)
        one_hot = (idx[:, :, None] == iota).astype(jnp.float32)
        one_hot2 = one_hot.reshape(B1 * 128, vocab)
        out_ref[...] = jax.lax.dot_general(
            one_hot2,
            table_ref[...],
            dimension_numbers=(((1,), (0,)), ((), ())),
            preferred_element_type=jnp.float32,
            precision=jax.lax.Precision.HIGHEST,
        )

    return pl.pallas_call(
        body,
        grid=(grid,),
        in_specs=[
            pl.BlockSpec((B1, 128), lambda i: (i, 0)),
            pl.BlockSpec((vocab, EMBD), lambda i: (0, 0)),
        ],
        out_specs=pl.BlockSpec((B1 * 128, EMBD), lambda i: (i, 0)),
        out_shape=jax.ShapeDtypeStruct((n_total, EMBD), jnp.float32),
    )


def kernel(vis, table):
    b, h = vis.shape
    n_total = b * h
    idx = vis.astype(jnp.int32).reshape(n_total // 128, 128)
    out = _make_tc(n_total, table.shape[0])(idx, table)
    return out.reshape(b, h, EMBD)


# speedup vs baseline: 2.1059x; 2.1059x over previous
import functools

import jax
import jax.numpy as jnp
from jax import lax
from jax.experimental import pallas as pl
from jax.experimental.pallas import tpu as pltpu
from jax.experimental.pallas import tpu_sc as plsc

EMBD = 128
CHUNK = 128
NBUF = 5
AHEAD = 3


@functools.cache
def _make_impl(n_total: int):
    info = plsc.get_sparse_core_info()
    nc, ns = info.num_cores, info.num_subcores
    nw = nc * ns
    n_per_w = n_total // nw
    n_chunks = n_per_w // CHUNK
    assert n_per_w * nw == n_total and n_chunks * CHUNK == n_per_w
    assert n_chunks % NBUF == 0

    mesh = plsc.VectorSubcoreMesh(core_axis_name="c", subcore_axis_name="s")

    @functools.partial(
        pl.kernel,
        out_type=jax.ShapeDtypeStruct((n_total, EMBD), jnp.float32),
        mesh=mesh,
        scratch_types=[
            pltpu.VMEM((n_chunks, CHUNK), jnp.int32),
            pltpu.VMEM((NBUF, CHUNK, EMBD), jnp.float32),
            pltpu.VMEM_SHARED((64, EMBD), jnp.float32),
            pltpu.SemaphoreType.DMA((NBUF,)),
            pltpu.SemaphoreType.DMA((NBUF,)),
        ],
    )
    def impl(idx_hbm, table_hbm, out_hbm, idx_v, rows_v, table_v, gsem, wsem):
        sid = lax.axis_index("s")
        wid = sid * nc + lax.axis_index("c")
        base = wid * n_per_w
        pltpu.sync_copy(idx_hbm.at[wid], idx_v)

        @pl.when(sid == 0)
        def _():
            pltpu.sync_copy(table_hbm, table_v)

        plsc.subcore_barrier()

        def gather(c, b):
            pltpu.async_copy(table_v.at[idx_v.at[c]], rows_v.at[b], gsem.at[b])

        def wait_gather(b):
            pltpu.make_async_copy(
                out_hbm.at[pl.ds(0, CHUNK)], rows_v.at[b], gsem.at[b]
            ).wait()

        def writeback(c, b):
            pltpu.async_copy(
                rows_v.at[b], out_hbm.at[pl.ds(base + c * CHUNK, CHUNK)], wsem.at[b]
            )

        def wait_writeback(b):
            pltpu.make_async_copy(
                rows_v.at[b], out_hbm.at[pl.ds(base, CHUNK)], wsem.at[b]
            ).wait()

        for b in range(NBUF):
            gather(b, b)

        def body(g, _):
            for b in range(NBUF):
                c = g * NBUF + b
                wait_gather(b)
                writeback(c, b)
                t = c + AHEAD
                tb = (b + AHEAD) % NBUF

                @pl.when(jnp.logical_and(t >= NBUF, t < n_chunks))
                def _():
                    wait_writeback(tb)
                    gather(t, tb)

            return _

        lax.fori_loop(0, n_chunks // NBUF, body, None)
        for b in range(NBUF):
            wait_writeback(b)

    return impl


def kernel(vis, table):
    b, h = vis.shape
    n_total = b * h
    info = plsc.get_sparse_core_info()
    nw = info.num_cores * info.num_subcores
    idx = vis.astype(jnp.int32).reshape(nw, (n_total // nw) // CHUNK, CHUNK)
    out = _make_impl(n_total)(idx, table)
    return out.reshape(b, h, EMBD)
